# baseline (speedup 1.0000x reference)
import math

import jax
import jax.numpy as jnp
from jax.experimental import pallas as pl
from jax.experimental.pallas import tpu as pltpu

NUM_BUCKETS = 320
MAX_DISTANCE = 1280
GATE_DIM = 8
N_VARIANTS = 16

_ROW_BLOCK = 512


def _materialize_body(
    table_t_ref, q_ref, w_ref, b_ref, gw_ref, attn_ref, pb_ref, bv_ref
):
    h = pl.program_id(0)
    r = pl.program_id(1)
    rows, seq = pb_ref.shape[1], pb_ref.shape[2]
    num_heads, _, _, l2 = bv_ref.shape

    @pl.when(jnp.logical_and(h == 0, r == 0))
    def _build_bias_table():
        num_buckets = NUM_BUCKETS // 2
        max_exact = num_buckets // 2
        k = jax.lax.broadcasted_iota(jnp.int32, (1, l2), 1)
        rel = k - (l2 // 2 - 1)
        pos_half = (rel > 0).astype(jnp.int32) * num_buckets
        a = jnp.abs(rel)
        af = a.astype(jnp.float32)
        large = max_exact + (
            jnp.log(jnp.maximum(af, 1.0) / max_exact)
            / math.log(MAX_DISTANCE / max_exact)
            * (num_buckets - max_exact)
        ).astype(jnp.int32)
        large = jnp.minimum(large, num_buckets - 1)
        bucket = pos_half + jnp.where(a < max_exact, a, large)

        b_iota = jax.lax.broadcasted_iota(jnp.int32, (NUM_BUCKETS, l2), 0)
        onehot_t = (b_iota == bucket).astype(jnp.float32)
        vt = jnp.dot(
            table_t_ref[...], onehot_t, preferred_element_type=jnp.float32
        )
        for u in range(N_VARIANTS):
            for s in range(8):
                d = 8 * u + s + 1
                shifted = jnp.concatenate(
                    [jnp.zeros((num_heads, d), jnp.float32), vt[:, : l2 - d]],
                    axis=1,
                )
                bv_ref[:, u, s, :] = shifted

    qb = q_ref[0, 0]
    proj = jax.lax.dot_general(
        qb, w_ref[...], (((1,), (1,)), ((), ())),
        preferred_element_type=jnp.float32,
    ) + b_ref[...]
    ga = jax.nn.sigmoid(jnp.sum(proj[:, : GATE_DIM // 2], axis=1, keepdims=True))
    gb = jax.nn.sigmoid(jnp.sum(proj[:, GATE_DIM // 2 :], axis=1, keepdims=True))
    lane = jax.lax.broadcasted_iota(jnp.int32, gw_ref.shape, 1)
    wh = jnp.sum(jnp.where(lane == h, gw_ref[...], 0.0))
    gate = ga * (gb * wh - 1.0) + 2.0

    base = pl.multiple_of(seq - rows * (r + 1), 128)
    for t in range(rows // 8):
        u = t % N_VARIANTS
        lo = rows - 128 * (t // N_VARIANTS)
        blk = bv_ref[h, u, :, pl.ds(base + lo, seq)]
        pb_ref[0, t * 8 : (t + 1) * 8, :] = blk
        attn_ref[0, t * 8 : (t + 1) * 8, :] = gate[t * 8 : (t + 1) * 8, :] * blk


def kernel(q, emb_table, grep_W, grep_b, grep_w):
    bsz, num_heads, seq_len, q_head_dim = q.shape
    l2 = 2 * seq_len

    rows = _ROW_BLOCK
    grid = (num_heads, seq_len // rows)
    attn, pb = pl.pallas_call(
        _materialize_body,
        grid=grid,
        in_specs=[
            pl.BlockSpec((num_heads, NUM_BUCKETS), lambda h, r: (0, 0)),
            pl.BlockSpec((1, 1, rows, q_head_dim), lambda h, r: (0, h, r, 0)),
            pl.BlockSpec((GATE_DIM, q_head_dim), lambda h, r: (0, 0)),
            pl.BlockSpec((1, GATE_DIM), lambda h, r: (0, 0)),
            pl.BlockSpec((1, num_heads), lambda h, r: (0, 0)),
        ],
        out_specs=[
            pl.BlockSpec((1, rows, seq_len), lambda h, r: (h, r, 0)),
            pl.BlockSpec((1, rows, seq_len), lambda h, r: (h, r, 0)),
        ],
        out_shape=[
            jax.ShapeDtypeStruct((bsz * num_heads, seq_len, seq_len), jnp.float32),
            jax.ShapeDtypeStruct((bsz * num_heads, seq_len, seq_len), jnp.float32),
        ],
        scratch_shapes=[pltpu.VMEM((num_heads, N_VARIANTS, 8, l2), jnp.float32)],
    )(
        emb_table.T,
        q,
        grep_W,
        grep_b.reshape(1, GATE_DIM),
        grep_w.reshape(1, num_heads),
    )

    return attn, pb


# speedup vs baseline: 1.0447x; 1.0447x over previous
import math

import jax
import jax.numpy as jnp
from jax.experimental import pallas as pl
from jax.experimental.pallas import tpu as pltpu

NUM_BUCKETS = 320
MAX_DISTANCE = 1280
GATE_DIM = 8

_ROW_BLOCK = 1024


def _materialize_body(
    table_t_ref, q_ref, w_ref, b_ref, gw_ref, attn_ref, pb_ref, bv_ref
):
    h = pl.program_id(0)
    r = pl.program_id(1)
    rows, seq = pb_ref.shape[1], pb_ref.shape[2]
    num_heads, _, l2 = bv_ref.shape

    @pl.when(jnp.logical_and(h == 0, r == 0))
    def _build_bias_table():
        num_buckets = NUM_BUCKETS // 2
        max_exact = num_buckets // 2
        k = jax.lax.broadcasted_iota(jnp.int32, (1, l2), 1)
        rel = k - (l2 // 2 - 1)
        pos_half = (rel > 0).astype(jnp.int32) * num_buckets
        a = jnp.abs(rel)
        af = a.astype(jnp.float32)
        large = max_exact + (
            jnp.log(jnp.maximum(af, 1.0) / max_exact)
            / math.log(MAX_DISTANCE / max_exact)
            * (num_buckets - max_exact)
        ).astype(jnp.int32)
        large = jnp.minimum(large, num_buckets - 1)
        bucket = pos_half + jnp.where(a < max_exact, a, large)

        b_iota = jax.lax.broadcasted_iota(jnp.int32, (NUM_BUCKETS, l2), 0)
        onehot_t = (b_iota == bucket).astype(jnp.float32)
        vt = jnp.dot(
            table_t_ref[...], onehot_t, preferred_element_type=jnp.float32
        )
        for s in range(8):
            shifted = jnp.concatenate(
                [jnp.zeros((num_heads, s + 1), jnp.float32), vt[:, : l2 - s - 1]],
                axis=1,
            )
            bv_ref[:, s, :] = shifted

    qb = q_ref[0, 0]
    proj = jax.lax.dot_general(
        qb, w_ref[...], (((1,), (1,)), ((), ())),
        preferred_element_type=jnp.float32,
    ) + b_ref[...]
    ga = jax.nn.sigmoid(jnp.sum(proj[:, : GATE_DIM // 2], axis=1, keepdims=True))
    gb = jax.nn.sigmoid(jnp.sum(proj[:, GATE_DIM // 2 :], axis=1, keepdims=True))
    lane = jax.lax.broadcasted_iota(jnp.int32, gw_ref.shape, 1)
    wh = jnp.sum(jnp.where(lane == h, gw_ref[...], 0.0))
    gate = ga * (gb * wh - 1.0) + 2.0

    ws = pl.multiple_of(seq - rows * (r + 1), 128)
    wide = bv_ref[h, :, pl.ds(ws, seq + rows)]
    for t in range(rows // 8):
        lo = rows - 8 * t
        blk = wide[:, lo : lo + seq]
        pb_ref[0, t * 8 : (t + 1) * 8, :] = blk
        attn_ref[0, t * 8 : (t + 1) * 8, :] = gate[t * 8 : (t + 1) * 8, :] * blk


def kernel(q, emb_table, grep_W, grep_b, grep_w):
    bsz, num_heads, seq_len, q_head_dim = q.shape
    l2 = 2 * seq_len

    rows = _ROW_BLOCK
    grid = (num_heads, seq_len // rows)
    attn, pb = pl.pallas_call(
        _materialize_body,
        grid=grid,
        in_specs=[
            pl.BlockSpec((num_heads, NUM_BUCKETS), lambda h, r: (0, 0)),
            pl.BlockSpec((1, 1, rows, q_head_dim), lambda h, r: (0, h, r, 0)),
            pl.BlockSpec((GATE_DIM, q_head_dim), lambda h, r: (0, 0)),
            pl.BlockSpec((1, GATE_DIM), lambda h, r: (0, 0)),
            pl.BlockSpec((1, num_heads), lambda h, r: (0, 0)),
        ],
        out_specs=[
            pl.BlockSpec((1, rows, seq_len), lambda h, r: (h, r, 0)),
            pl.BlockSpec((1, rows, seq_len), lambda h, r: (h, r, 0)),
        ],
        out_shape=[
            jax.ShapeDtypeStruct((bsz * num_heads, seq_len, seq_len), jnp.float32),
            jax.ShapeDtypeStruct((bsz * num_heads, seq_len, seq_len), jnp.float32),
        ],
        scratch_shapes=[pltpu.VMEM((num_heads, 8, l2), jnp.float32)],
    )(
        emb_table.T,
        q,
        grep_W,
        grep_b.reshape(1, GATE_DIM),
        grep_w.reshape(1, num_heads),
    )

    return attn, pb
